# Initial kernel scaffold; baseline (speedup 1.0000x reference)
import jax
import jax.numpy as jnp
from jax.experimental import pallas as pl


def kernel(x, neighbors, W1_nn, b1_nn, g_nn, be_nn, W2_nn, b2_nn, W_self, b_self, W1_out, b1_out, g_out, be_out, W2_out, b2_out):
    raise NotImplementedError("write your pallas kernel here")


# speedup vs baseline: 1.9739x; 1.9739x over previous
import functools

import jax
import jax.numpy as jnp
from jax.experimental import pallas as pl
from jax.experimental.pallas import tpu as pltpu


def _tc_body(scal_ref, c1_ref, c2_ref, ben_ref, n_ref, xT_ref, Wacc_ref,
             aux_ref, W2o_ref, out_ref, acc_ref):
    A = scal_ref[0]
    B2 = scal_ref[1]
    Ceps = scal_ref[2]
    V = n_ref[0]
    S = jax.lax.rsqrt((A * V + B2) * V + Ceps)

    def kbody(k, carry):
        c1k = c1_ref[k]
        c2k = c2_ref[k]
        bek = ben_ref[k]
        t = jnp.maximum((V * c1k + c2k) * S + bek, 0.0)
        acc_ref[pl.ds(k, 1), :] = jnp.sum(t, axis=0, keepdims=True)
        return carry

    jax.lax.fori_loop(0, 32, kbody, 0)

    acc = acc_ref[...]
    o1 = jax.lax.dot_general(acc, Wacc_ref[...],
                             (((0,), (0,)), ((), ())),
                             preferred_element_type=jnp.float32)
    o1 = o1 + xT_ref[0] * aux_ref[0:1, :] + aux_ref[1:2, :]
    m = jnp.mean(o1, axis=1, keepdims=True)
    var = jnp.mean((o1 - m) ** 2, axis=1, keepdims=True)
    o2 = (o1 - m) * jax.lax.rsqrt(var + 1e-5) * aux_ref[2:3, :] + aux_ref[3:4, :]
    o2 = jnp.maximum(o2, 0.0)
    out_ref[0] = jnp.dot(o2, W2o_ref[...],
                         preferred_element_type=jnp.float32) + aux_ref[4:5, :]


def kernel(x, neighbors, W1_nn, b1_nn, g_nn, be_nn, W2_nn, b2_nn,
           W_self, b_self, W1_out, b1_out, g_out, be_out, W2_out, b2_out):
    B, G = x.shape
    N = neighbors.shape[2]
    merge = W1_nn.shape[1]
    outd = W1_out.shape[1]

    w1 = W1_nn[0]
    mW = jnp.mean(w1)
    mb = jnp.mean(b1_nn)
    a = w1 - mW
    d = b1_nn - mb
    A = jnp.mean(a * a)
    Bc = jnp.mean(a * d)
    C = jnp.mean(d * d)
    c1 = a * g_nn
    c2 = d * g_nn
    Wacc = W2_nn @ W1_out[merge:]
    wx = W_self[0] @ W1_out[:merge]
    bias0 = b_self @ W1_out[:merge] + G * (b2_nn @ W1_out[merge:]) + b1_out
    scal = jnp.stack([A, 2.0 * Bc, C + 1e-5])
    aux = jnp.stack([wx, bias0, g_out, be_out, b2_out])
    xcol = x[..., None]

    grid = (B,)
    out = pl.pallas_call(
        _tc_body,
        grid=grid,
        in_specs=[
            pl.BlockSpec(memory_space=pltpu.SMEM),
            pl.BlockSpec(memory_space=pltpu.SMEM),
            pl.BlockSpec(memory_space=pltpu.SMEM),
            pl.BlockSpec(memory_space=pltpu.SMEM),
            pl.BlockSpec((1, G, N), lambda b: (b, 0, 0)),
            pl.BlockSpec((1, G, 1), lambda b: (b, 0, 0)),
            pl.BlockSpec((merge, outd), lambda b: (0, 0)),
            pl.BlockSpec((5, outd), lambda b: (0, 0)),
            pl.BlockSpec((outd, outd), lambda b: (0, 0)),
        ],
        out_specs=pl.BlockSpec((1, N, outd), lambda b: (b, 0, 0)),
        out_shape=jax.ShapeDtypeStruct((B, N, outd), jnp.float32),
        scratch_shapes=[pltpu.VMEM((merge, N), jnp.float32)],
    )(scal, c1, c2, be_nn, neighbors, xcol, Wacc, aux, W2_out)
    return out
